# baseline (speedup 1.0000x reference)
import functools

import jax
import jax.numpy as jnp
from jax import lax
from jax.experimental import pallas as pl
from jax.experimental.pallas import tpu as pltpu
from jax.experimental.pallas import tpu_sc as plsc

VOCAB = 100000
EMBED = 64
ROWS = 4096
COLS = 50
NUM_WORKERS = 32
ROWS_PW = ROWS // NUM_WORKERS
CHUNK_I = 2
CHUNK = CHUNK_I * COLS
NCHUNK = ROWS_PW // CHUNK_I
NBUF = 8
NSTEP = NCHUNK // NBUF

_MESH = plsc.VectorSubcoreMesh(core_axis_name="c", subcore_axis_name="s")


@functools.partial(
    pl.kernel,
    mesh=_MESH,
    out_type=jax.ShapeDtypeStruct((ROWS, COLS, EMBED), jnp.float32),
    scratch_types=[
        pltpu.VMEM((NCHUNK, CHUNK), jnp.int32),
        pltpu.VMEM((NBUF, CHUNK, EMBED), jnp.float32),
        pltpu.SemaphoreType.DMA((NBUF,)),
        pltpu.SemaphoreType.DMA((NBUF,)),
    ],
    compiler_params=pltpu.CompilerParams(use_tc_tiling_on_sc=False),
)
def _embed_gather(idx_hbm, table_hbm, out_hbm, idx_v, rows_v, gsem, ssem):
    wid = lax.axis_index("s") * 2 + lax.axis_index("c")
    i0 = wid * ROWS_PW
    pltpu.sync_copy(idx_hbm.at[wid], idx_v)

    def superstep(s, carry):
        c0 = s * NBUF
        gh = []
        for b in range(NBUF):
            gh.append(pltpu.async_copy(
                table_hbm.at[idx_v.at[c0 + b]], rows_v.at[b], gsem.at[b]))
        sh = []
        for b in range(NBUF):
            gh[b].wait()
            for t in range(CHUNK_I):
                sh.append(pltpu.async_copy(
                    rows_v.at[b].at[pl.ds(t * COLS, COLS)],
                    out_hbm.at[i0 + (c0 + b) * CHUNK_I + t],
                    ssem.at[b]))
        for h in sh:
            h.wait()
        return carry

    lax.fori_loop(0, NSTEP, superstep, 0)


def kernel(indices, weight):
    idx = indices.astype(jnp.int32).reshape(NUM_WORKERS, NCHUNK, CHUNK)
    return _embed_gather(idx, weight)
